# Initial kernel scaffold; baseline (speedup 1.0000x reference)
import jax
import jax.numpy as jnp
from jax.experimental import pallas as pl


def kernel(h, a, We, be, Wg, bg):
    raise NotImplementedError("write your pallas kernel here")


# speedup vs baseline: 2.2496x; 2.2496x over previous
import functools

import jax
import jax.numpy as jnp
from jax.experimental import pallas as pl
from jax.experimental.pallas import tpu as pltpu


def _moe_kernel(a_ref, wg_ref, bg_ref, h_ref, we_ref, be_ref,
                out_ref, ent_ref, *, bt, n_blocks, n_exp, d_model, seq_len,
                n_tokens):
    i = pl.program_id(0)

    h = h_ref[...]
    h_bf = h.astype(jnp.bfloat16)

    wg = wg_ref[...]
    wg_h = wg[:, :d_model].astype(jnp.bfloat16)
    wg_a = wg[:, d_model:].astype(jnp.bfloat16)
    b_idx = i * bt // seq_len
    a_bf = a_ref[pl.ds(b_idx, 1), :].astype(jnp.bfloat16)

    logits = jax.lax.dot_general(
        h_bf, wg_h, (((1,), (1,)), ((), ())),
        preferred_element_type=jnp.float32)
    logits_a = jax.lax.dot_general(
        a_bf, wg_a, (((1,), (1,)), ((), ())),
        preferred_element_type=jnp.float32)
    logits = logits + logits_a + bg_ref[...]

    m = jnp.max(logits, axis=-1, keepdims=True)
    p = jnp.exp(logits - m)
    s = jnp.sum(p, axis=-1, keepdims=True)
    g = p / s

    iota = jax.lax.broadcasted_iota(jnp.int32, g.shape, 1)
    m1 = jnp.max(g, axis=-1, keepdims=True)
    i1 = jnp.min(jnp.where(g == m1, iota, n_exp), axis=-1, keepdims=True)
    g_masked = jnp.where(iota == i1, -jnp.inf, g)
    m2 = jnp.max(g_masked, axis=-1, keepdims=True)
    i2 = jnp.min(jnp.where(g_masked == m2, iota, n_exp), axis=-1,
                 keepdims=True)
    denom = m1 + m2
    w = (jnp.where(iota == i1, m1, 0.0)
         + jnp.where(iota == i2, m2, 0.0)) / denom

    part = jnp.sum(g * jnp.log(g + 1e-10))
    @pl.when(i == 0)
    def _():
        ent_ref[0, 0] = jnp.float32(0.0)
    ent_ref[0, 0] += part
    @pl.when(i == n_blocks - 1)
    def _():
        ent_ref[0, 0] = ent_ref[0, 0] * jnp.float32(-1.0 / n_tokens)

    acc = jnp.zeros((bt, d_model), dtype=jnp.float32)
    for e in range(n_exp):
        we_bf = we_ref[e, :, :].astype(jnp.bfloat16)
        y = jax.lax.dot_general(
            h_bf, we_bf, (((1,), (1,)), ((), ())),
            preferred_element_type=jnp.float32)
        w_e = w[:, e:e + 1]
        acc = acc + w_e * (y + be_ref[e:e + 1, :])
    out_ref[...] = acc


@functools.partial(jax.jit, static_argnames=())
def kernel(h, a, We, be, Wg, bg):
    b, s, d = h.shape
    n_exp, _, _ = We.shape
    n_att = a.shape[-1]
    n_tokens = b * s
    bt = 512
    n_blocks = n_tokens // bt

    h2 = h.reshape(n_tokens, d)
    bg2 = bg.reshape(1, n_exp)

    kern = functools.partial(
        _moe_kernel, bt=bt, n_blocks=n_blocks, n_exp=n_exp, d_model=d,
        seq_len=s, n_tokens=n_tokens)

    out, ent = pl.pallas_call(
        kern,
        grid=(n_blocks,),
        in_specs=[
            pl.BlockSpec((b, n_att), lambda i: (0, 0)),
            pl.BlockSpec((n_exp, d + n_att), lambda i: (0, 0)),
            pl.BlockSpec((1, n_exp), lambda i: (0, 0)),
            pl.BlockSpec((bt, d), lambda i: (i, 0)),
            pl.BlockSpec((n_exp, d, d), lambda i: (0, 0, 0)),
            pl.BlockSpec((n_exp, d), lambda i: (0, 0)),
        ],
        out_specs=[
            pl.BlockSpec((bt, d), lambda i: (i, 0)),
            pl.BlockSpec(memory_space=pltpu.SMEM),
        ],
        out_shape=[
            jax.ShapeDtypeStruct((n_tokens, d), jnp.float32),
            jax.ShapeDtypeStruct((1, 1), jnp.float32),
        ],
        compiler_params=pltpu.CompilerParams(
            dimension_semantics=("arbitrary",),
        ),
    )(a, Wg, bg2, h2, We, be)

    output = out.reshape(b, s, d)
    entropy_loss = ent[0, 0]
    stability_loss = jnp.float32(0.0)
    return (output, entropy_loss, stability_loss)
